# Initial kernel scaffold; baseline (speedup 1.0000x reference)
import jax
import jax.numpy as jnp
from jax.experimental import pallas as pl


def kernel(x, edge_index, Y, us1_W, us1_b, usmu_W, usmu_b, uslog_W, uslog_b, uy1_W, uy1_b, uymu_W, uymu_b, uylog_W, uylog_b, sd1_W, sd1_b, sd2_W, sd2_b, xd1_W, xd1_b, xd2_W, xd2_b, fc1A_W, fc1A_b, fc2A_W, fc2A_b, yd1_W, yd1_b, yd2_W, yd2_b, yp1_W, yp1_b, yp2_W, yp2_b):
    raise NotImplementedError("write your pallas kernel here")


# speedup vs baseline: 9.9050x; 9.9050x over previous
import jax
import jax.numpy as jnp
import numpy as np
from jax.experimental import pallas as pl
from jax.experimental.pallas import tpu as pltpu

N = 512
E = 16384
F_IN = 128
H = 256
LS = 32
LY = 32
NL = 8
NUM_EDGES = N * (N - 1) // 2 + N

_INTERPRET = False


def _ahat_body(ei_ref, out_ref):
    iota_n = jax.lax.broadcasted_iota(jnp.int32, (N, 1), 0)
    C = jnp.zeros((N, N), jnp.float32)
    CHUNK = 2048
    for k in range(E // CHUNK):
        src = ei_ref[0:1, k * CHUNK:(k + 1) * CHUNK]
        dst = ei_ref[1:2, k * CHUNK:(k + 1) * CHUNK]
        osrc_t = (iota_n == src).astype(jnp.bfloat16)
        odst_t = (iota_n == dst).astype(jnp.bfloat16)
        C = C + jax.lax.dot_general(
            odst_t, osrc_t, (((1,), (1,)), ((), ())),
            preferred_element_type=jnp.float32)
    eye = (iota_n == jax.lax.broadcasted_iota(jnp.int32, (1, N), 1)
           ).astype(jnp.float32)
    C = C + eye
    deg = jnp.sum(C, axis=1, keepdims=True)
    dinv = jax.lax.rsqrt(deg)
    out_ref[...] = C * dinv * dinv.reshape(1, N)


def _build_ahat(edge_index):
    return pl.pallas_call(
        _ahat_body,
        out_shape=jax.ShapeDtypeStruct((N, N), jnp.float32),
        interpret=_INTERPRET,
    )(edge_index)


def _forward_body(x_ref, y_ref, ahat_ref, eps_s_ref, eps_y_ref,
                  us1w, us1b, usmuw, usmub, uslogw, uslogb,
                  uy1aw, uy1bw, uy1b, uymuw, uymub, uylogw, uylogb,
                  sd1w, sd1b, sd2w, sd2b,
                  xd1aw, xd1bw, xd1b, xd2w, xd2b,
                  yd1aw, yd1bw, yd1b, yd2w, yd2b,
                  yp1w, yp1b, yp2w, yp2b,
                  mu_s_o, log_s_o, mu_y_o, log_y_o, u_s_o, u_y_o,
                  s_hat_o, xp_o, ypred_o, yprime_o):
    A = ahat_ref[...]
    x = x_ref[...]

    def g(h, w_ref, b_ref):
        return jnp.dot(A, jnp.dot(h, w_ref[...],
                                  preferred_element_type=jnp.float32),
                       preferred_element_type=jnp.float32) + b_ref[...]

    h = jax.nn.relu(g(x, us1w, us1b))
    mu_s = g(h, usmuw, usmub)
    log_s = g(h, uslogw, uslogb)
    mu_s_o[...] = mu_s
    log_s_o[...] = log_s
    xw2 = jnp.dot(x, uy1aw[...], preferred_element_type=jnp.float32) \
        + y_ref[...] * uy1bw[...]
    h2 = jax.nn.relu(jnp.dot(A, xw2, preferred_element_type=jnp.float32)
                     + uy1b[...])
    mu_y = g(h2, uymuw, uymub)
    log_y = g(h2, uylogw, uylogb)
    mu_y_o[...] = mu_y
    log_y_o[...] = log_y
    u_s = eps_s_ref[...] * jnp.exp(0.5 * log_s) + mu_s
    u_y = eps_y_ref[...] * jnp.exp(0.5 * log_y) + mu_y
    u_s_o[...] = u_s
    u_y_o[...] = u_y
    s1 = jax.nn.relu(g(u_s, sd1w, sd1b))
    s2 = jax.nn.relu(g(s1, sd2w, sd2b))
    s_hat_o[...] = jax.nn.sigmoid(s2)
    xw3 = jnp.dot(u_s, xd1aw[...], preferred_element_type=jnp.float32) \
        + jnp.dot(u_y, xd1bw[...], preferred_element_type=jnp.float32)
    xp1 = jnp.dot(A, xw3, preferred_element_type=jnp.float32) + xd1b[...]
    xp = g(xp1, xd2w, xd2b)
    xp_o[...] = xp
    xw4 = jnp.dot(xp, yd1aw[...], preferred_element_type=jnp.float32) \
        + jnp.dot(u_y, yd1bw[...], preferred_element_type=jnp.float32)
    yl1 = jnp.dot(A, xw4, preferred_element_type=jnp.float32) + yd1b[...]
    ylog = g(yl1, yd2w, yd2b)
    ypred_o[...] = jax.nn.softmax(ylog, axis=1)
    yq1 = g(x, yp1w, yp1b)
    qlog = g(yq1, yp2w, yp2b)
    yprime_o[...] = jax.nn.softmax(qlog, axis=1)


def _run_forward(x, Y, ahat, eps_s, eps_y, params):
    outs = (
        jax.ShapeDtypeStruct((N, LS), jnp.float32),
        jax.ShapeDtypeStruct((N, LS), jnp.float32),
        jax.ShapeDtypeStruct((N, LY), jnp.float32),
        jax.ShapeDtypeStruct((N, LY), jnp.float32),
        jax.ShapeDtypeStruct((N, LS), jnp.float32),
        jax.ShapeDtypeStruct((N, LY), jnp.float32),
        jax.ShapeDtypeStruct((N, 128), jnp.float32),
        jax.ShapeDtypeStruct((N, F_IN), jnp.float32),
        jax.ShapeDtypeStruct((N, 128), jnp.float32),
        jax.ShapeDtypeStruct((N, 128), jnp.float32),
    )
    return pl.pallas_call(
        _forward_body,
        out_shape=outs,
        interpret=_INTERPRET,
    )(x, Y, ahat, eps_s, eps_y, *params)


def _fc1_body(f_ref, w_ref, b_ref, o_ref):
    @pl.when(pl.program_id(0) == 0)
    def _():
        o_ref[...] = b_ref[...]

    o_ref[...] += jnp.dot(f_ref[...], w_ref[...],
                          preferred_element_type=jnp.float32)


def _run_fc1(feat, W, b):
    KB = 4096
    return pl.pallas_call(
        _fc1_body,
        grid=(W.shape[0] // KB,),
        in_specs=[
            pl.BlockSpec((1, KB), lambda k: (0, k)),
            pl.BlockSpec((KB, 512), lambda k: (k, 0)),
            pl.BlockSpec((1, 512), lambda k: (0, 0)),
        ],
        out_specs=pl.BlockSpec((1, 512), lambda k: (0, 0)),
        out_shape=jax.ShapeDtypeStruct((1, 512), jnp.float32),
        interpret=_INTERPRET,
    )(feat, W, b)


def _fc2_body(v_ref, w_ref, b_ref, o_ref):
    o_ref[...] = jnp.dot(v_ref[...], w_ref[...],
                         preferred_element_type=jnp.float32) + b_ref[...]


def _run_fc2(v, W, b):
    CB = 2304
    return pl.pallas_call(
        _fc2_body,
        grid=(NUM_EDGES // CB,),
        in_specs=[
            pl.BlockSpec((1, 512), lambda j: (0, 0)),
            pl.BlockSpec((512, CB), lambda j: (0, j)),
            pl.BlockSpec((1, CB), lambda j: (0, j)),
        ],
        out_specs=pl.BlockSpec((1, CB), lambda j: (0, j)),
        out_shape=jax.ShapeDtypeStruct((1, NUM_EDGES), jnp.float32),
        interpret=_INTERPRET,
    )(v, W, b)


def _tri_body(l_ref, a_ref):
    iota_l = jax.lax.broadcasted_iota(jnp.int32, (1, N), 1)
    W = N + 128

    def body(i, _):
        start = i * (i + 1) // 2
        base = jnp.minimum((start // 128) * 128, NUM_EDGES - W)
        base = pl.multiple_of(base, 128)
        off = start - base
        w = l_ref[0:1, pl.ds(base, W)]
        row = pltpu.roll(w, (W - off) % W, axis=1)[:, :N]
        a_ref[pl.ds(i, 1), :] = jnp.where(iota_l <= i, row, 0.0)
        return 0

    jax.lax.fori_loop(0, N, body, 0)


def _run_tri(l2d):
    return pl.pallas_call(
        _tri_body,
        out_shape=jax.ShapeDtypeStruct((N, N), jnp.float32),
        interpret=_INTERPRET,
    )(l2d)


def kernel(x, edge_index, Y,
           us1_W, us1_b, usmu_W, usmu_b, uslog_W, uslog_b,
           uy1_W, uy1_b, uymu_W, uymu_b, uylog_W, uylog_b,
           sd1_W, sd1_b, sd2_W, sd2_b,
           xd1_W, xd1_b, xd2_W, xd2_b,
           fc1A_W, fc1A_b, fc2A_W, fc2A_b,
           yd1_W, yd1_b, yd2_W, yd2_b,
           yp1_W, yp1_b, yp2_W, yp2_b):
    f32 = jnp.float32
    row = lambda b: b.reshape(1, -1)
    sd2w_p = jnp.zeros((H, 128), f32).at[:, :1].set(sd2_W)
    sd2b_p = jnp.zeros((1, 128), f32).at[:, :1].set(row(sd2_b))
    neg = jnp.full((1, 128), -1e30, f32)
    yd2w_p = jnp.zeros((512, 128), f32).at[:, :NL].set(yd2_W)
    yd2b_p = neg.at[:, :NL].set(row(yd2_b))
    yp2w_p = jnp.zeros((512, 128), f32).at[:, :NL].set(yp2_W)
    yp2b_p = neg.at[:, :NL].set(row(yp2_b))

    eps_s = jax.random.normal(jax.random.key(42), (N, LS), dtype=f32)
    eps_y = jax.random.normal(jax.random.key(43), (N, LY), dtype=f32)

    ahat = _build_ahat(edge_index)

    params = (
        us1_W, row(us1_b), usmu_W, row(usmu_b), uslog_W, row(uslog_b),
        uy1_W[:F_IN], uy1_W[F_IN:F_IN + 1], row(uy1_b),
        uymu_W, row(uymu_b), uylog_W, row(uylog_b),
        sd1_W, row(sd1_b), sd2w_p, sd2b_p,
        xd1_W[:LS], xd1_W[LS:], row(xd1_b), xd2_W, row(xd2_b),
        yd1_W[:F_IN], yd1_W[F_IN:], row(yd1_b), yd2w_p, yd2b_p,
        yp1_W, row(yp1_b), yp2w_p, yp2b_p,
    )
    (mu_s, log_s, mu_y, log_y, u_s, u_y,
     s_hat_p, xp, ypred_p, yprime_p) = _run_forward(
        x, Y, ahat, eps_s, eps_y, params)

    feat = jnp.concatenate([u_s, u_y], axis=1).reshape(1, N * (LS + LY))
    l1 = _run_fc1(feat, fc1A_W, row(fc1A_b))
    l2d = _run_fc2(l1, fc2A_W, row(fc2A_b))
    l = l2d.reshape(NUM_EDGES)
    A = _run_tri(l2d)

    return (xp, A, l, ypred_p[:, :NL], yprime_p[:, :NL],
            s_hat_p[:, :1], mu_s, log_s, mu_y, log_y)


# speedup vs baseline: 10.0775x; 1.0174x over previous
import jax
import jax.numpy as jnp
import numpy as np
from jax.experimental import pallas as pl
from jax.experimental.pallas import tpu as pltpu

N = 512
E = 16384
F_IN = 128
H = 256
LS = 32
LY = 32
NL = 8
NUM_EDGES = N * (N - 1) // 2 + N

_INTERPRET = False


def _ahat_body(ei_ref, out_ref):
    iota_n = jax.lax.broadcasted_iota(jnp.int32, (N, 1), 0)
    C = jnp.zeros((N, N), jnp.float32)
    CHUNK = 2048
    for k in range(E // CHUNK):
        src = ei_ref[0:1, k * CHUNK:(k + 1) * CHUNK]
        dst = ei_ref[1:2, k * CHUNK:(k + 1) * CHUNK]
        osrc_t = (iota_n == src).astype(jnp.bfloat16)
        odst_t = (iota_n == dst).astype(jnp.bfloat16)
        C = C + jax.lax.dot_general(
            odst_t, osrc_t, (((1,), (1,)), ((), ())),
            preferred_element_type=jnp.float32)
    eye = (iota_n == jax.lax.broadcasted_iota(jnp.int32, (1, N), 1)
           ).astype(jnp.float32)
    C = C + eye
    deg = jnp.sum(C, axis=1, keepdims=True)
    dinv = jax.lax.rsqrt(deg)
    out_ref[...] = C * dinv * dinv.reshape(1, N)


def _build_ahat(edge_index):
    return pl.pallas_call(
        _ahat_body,
        out_shape=jax.ShapeDtypeStruct((N, N), jnp.float32),
        interpret=_INTERPRET,
    )(edge_index)


def _forward_body(x_ref, y_ref, ahat_ref, eps_s_ref, eps_y_ref,
                  us1w, us1b, usmuw, usmub, uslogw, uslogb,
                  uy1aw, uy1bw, uy1b, uymuw, uymub, uylogw, uylogb,
                  sd1w, sd1b, sd2w, sd2b,
                  xd1aw, xd1bw, xd1b, xd2w, xd2b,
                  yd1aw, yd1bw, yd1b, yd2w, yd2b,
                  yp1w, yp1b, yp2w, yp2b,
                  mu_s_o, log_s_o, mu_y_o, log_y_o, u_s_o, u_y_o,
                  s_hat_o, xp_o, ypred_o, yprime_o):
    A = ahat_ref[...]
    x = x_ref[...]

    def g(h, w_ref, b_ref):
        return jnp.dot(A, jnp.dot(h, w_ref[...],
                                  preferred_element_type=jnp.float32),
                       preferred_element_type=jnp.float32) + b_ref[...]

    h = jax.nn.relu(g(x, us1w, us1b))
    mu_s = g(h, usmuw, usmub)
    log_s = g(h, uslogw, uslogb)
    mu_s_o[...] = mu_s
    log_s_o[...] = log_s
    xw2 = jnp.dot(x, uy1aw[...], preferred_element_type=jnp.float32) \
        + y_ref[...] * uy1bw[...]
    h2 = jax.nn.relu(jnp.dot(A, xw2, preferred_element_type=jnp.float32)
                     + uy1b[...])
    mu_y = g(h2, uymuw, uymub)
    log_y = g(h2, uylogw, uylogb)
    mu_y_o[...] = mu_y
    log_y_o[...] = log_y
    u_s = eps_s_ref[...] * jnp.exp(0.5 * log_s) + mu_s
    u_y = eps_y_ref[...] * jnp.exp(0.5 * log_y) + mu_y
    u_s_o[...] = u_s
    u_y_o[...] = u_y
    s1 = jax.nn.relu(g(u_s, sd1w, sd1b))
    s2 = jax.nn.relu(g(s1, sd2w, sd2b))
    s_hat_o[...] = jax.nn.sigmoid(s2)
    xw3 = jnp.dot(u_s, xd1aw[...], preferred_element_type=jnp.float32) \
        + jnp.dot(u_y, xd1bw[...], preferred_element_type=jnp.float32)
    xp1 = jnp.dot(A, xw3, preferred_element_type=jnp.float32) + xd1b[...]
    xp = g(xp1, xd2w, xd2b)
    xp_o[...] = xp
    xw4 = jnp.dot(xp, yd1aw[...], preferred_element_type=jnp.float32) \
        + jnp.dot(u_y, yd1bw[...], preferred_element_type=jnp.float32)
    yl1 = jnp.dot(A, xw4, preferred_element_type=jnp.float32) + yd1b[...]
    ylog = g(yl1, yd2w, yd2b)
    ypred_o[...] = jax.nn.softmax(ylog, axis=1)
    yq1 = g(x, yp1w, yp1b)
    qlog = g(yq1, yp2w, yp2b)
    yprime_o[...] = jax.nn.softmax(qlog, axis=1)


def _run_forward(x, Y, ahat, eps_s, eps_y, params):
    outs = (
        jax.ShapeDtypeStruct((N, LS), jnp.float32),
        jax.ShapeDtypeStruct((N, LS), jnp.float32),
        jax.ShapeDtypeStruct((N, LY), jnp.float32),
        jax.ShapeDtypeStruct((N, LY), jnp.float32),
        jax.ShapeDtypeStruct((N, LS), jnp.float32),
        jax.ShapeDtypeStruct((N, LY), jnp.float32),
        jax.ShapeDtypeStruct((N, 128), jnp.float32),
        jax.ShapeDtypeStruct((N, F_IN), jnp.float32),
        jax.ShapeDtypeStruct((N, 128), jnp.float32),
        jax.ShapeDtypeStruct((N, 128), jnp.float32),
    )
    return pl.pallas_call(
        _forward_body,
        out_shape=outs,
        interpret=_INTERPRET,
    )(x, Y, ahat, eps_s, eps_y, *params)


def _fc1_body(f_ref, w_ref, b_ref, o_ref):
    @pl.when(pl.program_id(0) == 0)
    def _():
        o_ref[...] = b_ref[...]

    o_ref[...] += jnp.dot(f_ref[...], w_ref[...],
                          preferred_element_type=jnp.float32)


def _run_fc1(feat, W, b):
    KB = 8192
    return pl.pallas_call(
        _fc1_body,
        grid=(W.shape[0] // KB,),
        in_specs=[
            pl.BlockSpec((1, KB), lambda k: (0, k)),
            pl.BlockSpec((KB, 512), lambda k: (k, 0)),
            pl.BlockSpec((1, 512), lambda k: (0, 0)),
        ],
        out_specs=pl.BlockSpec((1, 512), lambda k: (0, 0)),
        out_shape=jax.ShapeDtypeStruct((1, 512), jnp.float32),
        interpret=_INTERPRET,
    )(feat, W, b)


def _fc2_body(v_ref, w_ref, b_ref, o_ref):
    o_ref[...] = jnp.dot(v_ref[...], w_ref[...],
                         preferred_element_type=jnp.float32) + b_ref[...]


def _run_fc2(v, W, b):
    CB = 6912
    return pl.pallas_call(
        _fc2_body,
        grid=(NUM_EDGES // CB,),
        in_specs=[
            pl.BlockSpec((1, 512), lambda j: (0, 0)),
            pl.BlockSpec((512, CB), lambda j: (0, j)),
            pl.BlockSpec((1, CB), lambda j: (0, j)),
        ],
        out_specs=pl.BlockSpec((1, CB), lambda j: (0, j)),
        out_shape=jax.ShapeDtypeStruct((1, NUM_EDGES), jnp.float32),
        interpret=_INTERPRET,
    )(v, W, b)


def _tri_body(l_ref, a_ref):
    iota_l = jax.lax.broadcasted_iota(jnp.int32, (1, N), 1)
    W = N + 128

    def body(i, _):
        start = i * (i + 1) // 2
        base = jnp.minimum((start // 128) * 128, NUM_EDGES - W)
        base = pl.multiple_of(base, 128)
        off = start - base
        w = l_ref[0:1, pl.ds(base, W)]
        row = pltpu.roll(w, (W - off) % W, axis=1)[:, :N]
        a_ref[pl.ds(i, 1), :] = jnp.where(iota_l <= i, row, 0.0)
        return 0

    jax.lax.fori_loop(0, N, body, 0)


def _run_tri(l2d):
    return pl.pallas_call(
        _tri_body,
        out_shape=jax.ShapeDtypeStruct((N, N), jnp.float32),
        interpret=_INTERPRET,
    )(l2d)


def kernel(x, edge_index, Y,
           us1_W, us1_b, usmu_W, usmu_b, uslog_W, uslog_b,
           uy1_W, uy1_b, uymu_W, uymu_b, uylog_W, uylog_b,
           sd1_W, sd1_b, sd2_W, sd2_b,
           xd1_W, xd1_b, xd2_W, xd2_b,
           fc1A_W, fc1A_b, fc2A_W, fc2A_b,
           yd1_W, yd1_b, yd2_W, yd2_b,
           yp1_W, yp1_b, yp2_W, yp2_b):
    f32 = jnp.float32
    row = lambda b: b.reshape(1, -1)
    sd2w_p = jnp.zeros((H, 128), f32).at[:, :1].set(sd2_W)
    sd2b_p = jnp.zeros((1, 128), f32).at[:, :1].set(row(sd2_b))
    neg = jnp.full((1, 128), -1e30, f32)
    yd2w_p = jnp.zeros((512, 128), f32).at[:, :NL].set(yd2_W)
    yd2b_p = neg.at[:, :NL].set(row(yd2_b))
    yp2w_p = jnp.zeros((512, 128), f32).at[:, :NL].set(yp2_W)
    yp2b_p = neg.at[:, :NL].set(row(yp2_b))

    eps_s = jax.random.normal(jax.random.key(42), (N, LS), dtype=f32)
    eps_y = jax.random.normal(jax.random.key(43), (N, LY), dtype=f32)

    ahat = _build_ahat(edge_index)

    params = (
        us1_W, row(us1_b), usmu_W, row(usmu_b), uslog_W, row(uslog_b),
        uy1_W[:F_IN], uy1_W[F_IN:F_IN + 1], row(uy1_b),
        uymu_W, row(uymu_b), uylog_W, row(uylog_b),
        sd1_W, row(sd1_b), sd2w_p, sd2b_p,
        xd1_W[:LS], xd1_W[LS:], row(xd1_b), xd2_W, row(xd2_b),
        yd1_W[:F_IN], yd1_W[F_IN:], row(yd1_b), yd2w_p, yd2b_p,
        yp1_W, row(yp1_b), yp2w_p, yp2b_p,
    )
    (mu_s, log_s, mu_y, log_y, u_s, u_y,
     s_hat_p, xp, ypred_p, yprime_p) = _run_forward(
        x, Y, ahat, eps_s, eps_y, params)

    feat = jnp.concatenate([u_s, u_y], axis=1).reshape(1, N * (LS + LY))
    l1 = _run_fc1(feat, fc1A_W, row(fc1A_b))
    l2d = _run_fc2(l1, fc2A_W, row(fc2A_b))
    l = l2d.reshape(NUM_EDGES)
    A = _run_tri(l2d)

    return (xp, A, l, ypred_p[:, :NL], yprime_p[:, :NL],
            s_hat_p[:, :1], mu_s, log_s, mu_y, log_y)


# speedup vs baseline: 13.5709x; 1.3467x over previous
import jax
import jax.numpy as jnp
import numpy as np
from jax.experimental import pallas as pl
from jax.experimental.pallas import tpu as pltpu

N = 512
E = 16384
F_IN = 128
H = 256
LS = 32
LY = 32
NL = 8
NUM_EDGES = N * (N - 1) // 2 + N

KB1 = 4096
P1 = (N * (LS + LY)) // KB1
CB2 = 6912
P2 = NUM_EDGES // CB2
WIN = N + 128

_INTERPRET = False


def _tri_schedule():
    tri = lambda i: i * (i + 1) // 2
    avail, i = [], 0
    for b in range(P2):
        hi = CB2 * (b + 1)
        while i < N and tri(i) + i + 1 <= hi:
            i += 1
        avail.append(i)
    cum, prev = [], 0
    for b in range(P2):
        prev = min(avail[b], max(prev, 27 * (b + 1)))
        cum.append(prev)
    cum[-1] = N
    return np.array([0] * (P1 + 1) + cum, dtype=np.int32)


def _fwd_body(ei_ref, x_ref, y_ref, eps_s_ref, eps_y_ref,
              us1w, us1b, usmuw, usmub, uslogw, uslogb,
              uy1w, uy1b, uymuw, uymub, uylogw, uylogb,
              sd1w, sd1b, sd2w, sd2b,
              xd1w, xd1b, xd2w, xd2b,
              yd1w, yd1b, yd2w, yd2b,
              yp1w, yp1b, yp2w, yp2b,
              mu_s_o, log_s_o, mu_y_o, log_y_o,
              s_hat_o, xp_o, ypred_o, yprime_o, lat_o):
    f32 = jnp.float32
    iota_n = jax.lax.broadcasted_iota(jnp.int32, (N, 1), 0)
    C = jnp.zeros((N, N), f32)
    CHUNK = 2048
    for k in range(E // CHUNK):
        src = ei_ref[0:1, k * CHUNK:(k + 1) * CHUNK]
        dst = ei_ref[1:2, k * CHUNK:(k + 1) * CHUNK]
        osrc_t = (iota_n == src).astype(jnp.bfloat16)
        odst_t = (iota_n == dst).astype(jnp.bfloat16)
        C = C + jax.lax.dot_general(
            odst_t, osrc_t, (((1,), (1,)), ((), ())),
            preferred_element_type=f32)
    eye = (iota_n == jax.lax.broadcasted_iota(jnp.int32, (1, N), 1)
           ).astype(f32)
    C = C + eye
    deg = jnp.sum(C, axis=1, keepdims=True)
    dinv = jax.lax.rsqrt(deg)
    A = C * dinv * dinv.reshape(1, N)

    x = x_ref[...]

    def mm(a, b):
        return jnp.dot(a, b, preferred_element_type=f32)

    def g(h, w_ref, b_ref):
        return mm(A, mm(h, w_ref[...])) + b_ref[...]

    h = jax.nn.relu(g(x, us1w, us1b))
    mu_s = g(h, usmuw, usmub)
    log_s = g(h, uslogw, uslogb)
    mu_s_o[...] = mu_s
    log_s_o[...] = log_s
    xw2 = mm(x, uy1w[0:F_IN, :]) + y_ref[...] * uy1w[F_IN:F_IN + 1, :]
    h2 = jax.nn.relu(mm(A, xw2) + uy1b[...])
    mu_y = g(h2, uymuw, uymub)
    log_y = g(h2, uylogw, uylogb)
    mu_y_o[...] = mu_y
    log_y_o[...] = log_y
    u_s = eps_s_ref[...] * jnp.exp(0.5 * log_s) + mu_s
    u_y = eps_y_ref[...] * jnp.exp(0.5 * log_y) + mu_y
    lat_o[...] = jnp.concatenate([u_s, u_y], axis=1)
    s1 = jax.nn.relu(g(u_s, sd1w, sd1b))
    s2 = jax.nn.relu(g(s1, sd2w, sd2b))
    s_hat_o[...] = jax.nn.sigmoid(s2)
    xw3 = mm(u_s, xd1w[0:LS, :]) + mm(u_y, xd1w[LS:LS + LY, :])
    xp1 = mm(A, xw3) + xd1b[...]
    xp = g(xp1, xd2w, xd2b)
    xp_o[...] = xp
    xw4 = mm(xp, yd1w[0:F_IN, :]) + mm(u_y, yd1w[F_IN:F_IN + LY, :])
    yl1 = mm(A, xw4) + yd1b[...]
    ypred_o[...] = jax.nn.softmax(g(yl1, yd2w, yd2b), axis=1)
    yq1 = g(x, yp1w, yp1b)
    yprime_o[...] = jax.nn.softmax(g(yq1, yp2w, yp2b), axis=1)


def _run_fwd(edge_index, x, Y, eps_s, eps_y, params):
    f32 = jnp.float32
    outs = (
        jax.ShapeDtypeStruct((N, LS), f32),
        jax.ShapeDtypeStruct((N, LS), f32),
        jax.ShapeDtypeStruct((N, LY), f32),
        jax.ShapeDtypeStruct((N, LY), f32),
        jax.ShapeDtypeStruct((N, 1), f32),
        jax.ShapeDtypeStruct((N, F_IN), f32),
        jax.ShapeDtypeStruct((N, NL), f32),
        jax.ShapeDtypeStruct((N, NL), f32),
        jax.ShapeDtypeStruct((N, LS + LY), f32),
    )
    return pl.pallas_call(
        _fwd_body,
        out_shape=outs,
        interpret=_INTERPRET,
    )(edge_index, x, Y, eps_s, eps_y, *params)


def _fc_body(sched_ref, f_ref, w1_ref, b1_ref, w2_ref, b2_ref,
             l_ref, a_ref, l1_ref):
    j = pl.program_id(0)

    @pl.when(j == 0)
    def _():
        l1_ref[...] = b1_ref[...]

    @pl.when(j < P1)
    def _():
        l1_ref[...] += jnp.dot(f_ref[...], w1_ref[...],
                               preferred_element_type=jnp.float32)

    @pl.when(j >= P1)
    def _():
        b = j - P1
        val = jnp.dot(l1_ref[...], w2_ref[...],
                      preferred_element_type=jnp.float32) + b2_ref[...]
        l_ref[0:1, pl.ds(pl.multiple_of(b * CB2, 128), CB2)] = val
        iota_l = jax.lax.broadcasted_iota(jnp.int32, (1, N), 1)

        def rowbody(i, _):
            start = i * (i + 1) // 2
            base = jnp.minimum((start // 128) * 128, NUM_EDGES - WIN)
            base = pl.multiple_of(base, 128)
            off = start - base
            w = l_ref[0:1, pl.ds(base, WIN)]
            rowv = pltpu.roll(w, (WIN - off) % WIN, axis=1)[:, :N]
            a_ref[pl.ds(i, 1), :] = jnp.where(iota_l <= i, rowv, 0.0)
            return 0

        jax.lax.fori_loop(sched_ref[j], sched_ref[j + 1], rowbody, 0)


def _run_fc(sched, feat, W1, b1, W2, b2):
    f32 = jnp.float32
    return pl.pallas_call(
        _fc_body,
        grid=(P1 + P2,),
        in_specs=[
            pl.BlockSpec(memory_space=pltpu.SMEM),
            pl.BlockSpec((1, KB1), lambda j: (0, jnp.minimum(j, P1 - 1))),
            pl.BlockSpec((KB1, 512), lambda j: (jnp.minimum(j, P1 - 1), 0)),
            pl.BlockSpec((1, 512), lambda j: (0, 0)),
            pl.BlockSpec((512, CB2),
                         lambda j: (0, jnp.clip(j - P1, 0, P2 - 1))),
            pl.BlockSpec((1, CB2),
                         lambda j: (0, jnp.clip(j - P1, 0, P2 - 1))),
        ],
        out_specs=(
            pl.BlockSpec((1, NUM_EDGES), lambda j: (0, 0)),
            pl.BlockSpec((N, N), lambda j: (0, 0)),
        ),
        out_shape=(
            jax.ShapeDtypeStruct((1, NUM_EDGES), f32),
            jax.ShapeDtypeStruct((N, N), f32),
        ),
        scratch_shapes=[pltpu.VMEM((1, 512), f32)],
        interpret=_INTERPRET,
    )(sched, feat, W1, b1, W2, b2)


def kernel(x, edge_index, Y,
           us1_W, us1_b, usmu_W, usmu_b, uslog_W, uslog_b,
           uy1_W, uy1_b, uymu_W, uymu_b, uylog_W, uylog_b,
           sd1_W, sd1_b, sd2_W, sd2_b,
           xd1_W, xd1_b, xd2_W, xd2_b,
           fc1A_W, fc1A_b, fc2A_W, fc2A_b,
           yd1_W, yd1_b, yd2_W, yd2_b,
           yp1_W, yp1_b, yp2_W, yp2_b):
    f32 = jnp.float32
    row = lambda b: b.reshape(1, -1)
    eps_s = jax.random.normal(jax.random.key(42), (N, LS), dtype=f32)
    eps_y = jax.random.normal(jax.random.key(43), (N, LY), dtype=f32)

    params = (
        us1_W, row(us1_b), usmu_W, row(usmu_b), uslog_W, row(uslog_b),
        uy1_W, row(uy1_b), uymu_W, row(uymu_b), uylog_W, row(uylog_b),
        sd1_W, row(sd1_b), sd2_W, row(sd2_b),
        xd1_W, row(xd1_b), xd2_W, row(xd2_b),
        yd1_W, row(yd1_b), yd2_W, row(yd2_b),
        yp1_W, row(yp1_b), yp2_W, row(yp2_b),
    )
    (mu_s, log_s, mu_y, log_y, s_hat, xp, ypred, yprime, lat) = _run_fwd(
        edge_index, x, Y, eps_s, eps_y, params)

    feat = lat.reshape(1, N * (LS + LY))
    sched = jnp.asarray(_tri_schedule())
    l2d, A = _run_fc(sched, feat, fc1A_W, row(fc1A_b), fc2A_W, row(fc2A_b))
    return (xp, A, l2d.reshape(NUM_EDGES), ypred, yprime,
            s_hat, mu_s, log_s, mu_y, log_y)
